# baseline (speedup 1.0000x reference)
import functools

import jax
import jax.numpy as jnp
from jax import lax
from jax.experimental import pallas as pl
from jax.experimental.pallas import tpu as pltpu
from jax.experimental.pallas import tpu_sc as plsc

_BW = 8192
_BB = 2048


def _transpose_pack_kernel(x_ref, o_ref):
    y = x_ref[...].T
    y3 = y.reshape(_BW // 4, 4, y.shape[1])
    o_ref[...] = jnp.concatenate([y3[:, k, :] for k in range(4)], axis=1)


def _tc_transpose_table(tt):
    d, v = tt.shape
    return pl.pallas_call(
        _transpose_pack_kernel,
        grid=(pl.cdiv(v, _BW),),
        in_specs=[pl.BlockSpec((d, _BW), lambda i: (0, i))],
        out_specs=pl.BlockSpec((_BW // 4, 128), lambda i: (i, 0)),
        out_shape=jax.ShapeDtypeStruct((v * d // 128, 128), jnp.float32),
    )(tt)


def _unpack_transpose_kernel(x_ref, o_ref):
    x = x_ref[...]
    d = o_ref.shape[1]
    z = jnp.stack([x[:, d * k : d * (k + 1)] for k in range(4)], axis=1)
    o_ref[0] = z.reshape(_BB, d).T


def _tc_detranspose_out(g2, f, b, d):
    nb = b // _BB
    return pl.pallas_call(
        _unpack_transpose_kernel,
        grid=(f, nb),
        in_specs=[
            pl.BlockSpec((_BB // 4, 128), lambda i, j, nb=nb: (i * nb + j, 0))
        ],
        out_specs=pl.BlockSpec((1, d, _BB), lambda i, j: (i, 0, j)),
        out_shape=jax.ShapeDtypeStruct((f, d, b), jnp.float32),
    )(g2)


def _make_sc_gather(b_total: int, d: int):
    info = plsc.get_sparse_core_info()
    nw = info.num_cores * info.num_subcores
    b_per_w = b_total // nw
    chunk = 1024
    n_chunks = b_per_w // chunk

    mesh = plsc.VectorSubcoreMesh(core_axis_name="c", subcore_axis_name="s")

    @functools.partial(
        pl.kernel,
        mesh=mesh,
        out_type=jax.ShapeDtypeStruct((b_total, d), jnp.float32),
        scratch_types=[
            pltpu.VMEM((b_per_w,), jnp.int32),
            pltpu.VMEM((2, chunk, d), jnp.float32),
            pltpu.SemaphoreType.DMA,
            pltpu.SemaphoreType.DMA,
        ],
        compiler_params=pltpu.CompilerParams(use_tc_tiling_on_sc=False),
    )
    def gather_kernel(table_hbm, idx_hbm, out_hbm, idx_v, rows_v, gsem, osem):
        wid = lax.axis_index("s") * info.num_cores + lax.axis_index("c")
        base = wid * b_per_w
        pltpu.sync_copy(idx_hbm.at[pl.ds(base, b_per_w)], idx_v)
        gathers = [None, None]
        outs = [None, None]
        gathers[0] = pltpu.async_copy(
            table_hbm.at[idx_v.at[pl.ds(0, chunk)]], rows_v.at[0], gsem
        )
        for c in range(n_chunks):
            cur = c % 2
            nxt = (c + 1) % 2
            if c + 1 < n_chunks:
                gathers[nxt] = pltpu.async_copy(
                    table_hbm.at[idx_v.at[pl.ds((c + 1) * chunk, chunk)]],
                    rows_v.at[nxt],
                    gsem,
                )
            gathers[cur].wait()
            if outs[cur] is not None:
                outs[cur].wait()
            outs[cur] = pltpu.async_copy(
                rows_v.at[cur],
                out_hbm.at[pl.ds(base + c * chunk, chunk)],
                osem,
            )
        for o in outs:
            if o is not None:
                o.wait()

    return gather_kernel


def kernel(inputs, indices):
    v, d = inputs.shape
    batch, n_fields = indices.shape
    table_packed = _tc_transpose_table(inputs.T)
    table_rm = table_packed.reshape(v, d)
    idx_flat = indices.T.reshape(-1)
    gathered = _make_sc_gather(idx_flat.shape[0], d)(table_rm, idx_flat)
    g2 = gathered.reshape(gathered.size // 128, 128)
    out_t = _tc_detranspose_out(g2, n_fields, batch, d)
    return out_t.transpose(2, 0, 1)


# speedup vs baseline: 1.8242x; 1.8242x over previous
import functools

import jax
import jax.numpy as jnp
from jax import lax
from jax.experimental import pallas as pl
from jax.experimental.pallas import tpu as pltpu
from jax.experimental.pallas import tpu_sc as plsc

_BW = 8192
_BB = 2048
_CK = 512


def _pack4_transpose_kernel(x_ref, o_ref):
    q = x_ref.shape[1] // 4
    d = x_ref.shape[0]
    for k in range(4):
        o_ref[:, d * k : d * (k + 1)] = x_ref[:, q * k : q * (k + 1)].T


def _tc_transpose_table(tt):
    d, v = tt.shape
    nb = pl.cdiv(v, _BW)
    return pl.pallas_call(
        _pack4_transpose_kernel,
        grid=(nb,),
        in_specs=[pl.BlockSpec((d, _BW), lambda i: (0, i))],
        out_specs=pl.BlockSpec((_BW // 4, 128), lambda i: (i, 0)),
        out_shape=jax.ShapeDtypeStruct((nb * _BW // 4, 128), jnp.float32),
    )(tt)


def _unpack4_transpose_kernel(x_ref, o_ref):
    d = o_ref.shape[1]
    q = x_ref.shape[0]
    for k in range(4):
        o_ref[0, :, q * k : q * (k + 1)] = x_ref[:, d * k : d * (k + 1)].T


def _tc_detranspose_out(g2, f, b, d):
    nb = b // _BB
    return pl.pallas_call(
        _unpack4_transpose_kernel,
        grid=(f, nb),
        in_specs=[
            pl.BlockSpec((_BB // 4, 128), lambda i, j, nb=nb: (i * nb + j, 0))
        ],
        out_specs=pl.BlockSpec((1, d, _BB), lambda i, j: (i, 0, j)),
        out_shape=jax.ShapeDtypeStruct((f, d, b), jnp.float32),
    )(g2)


def _make_sc_gather(b_total: int, batch: int, d: int):
    info = plsc.get_sparse_core_info()
    nw = info.num_cores * info.num_subcores
    b_per_w = b_total // nw
    n_chunks = b_per_w // _CK

    mesh = plsc.VectorSubcoreMesh(core_axis_name="c", subcore_axis_name="s")

    @functools.partial(
        pl.kernel,
        mesh=mesh,
        out_type=jax.ShapeDtypeStruct((b_total // 4, 128), jnp.float32),
        scratch_types=[
            pltpu.VMEM((b_per_w,), jnp.int32),
            pltpu.VMEM((2, _CK, d), jnp.float32),
            pltpu.SemaphoreType.DMA,
            pltpu.SemaphoreType.DMA,
        ],
        compiler_params=pltpu.CompilerParams(use_tc_tiling_on_sc=False),
    )
    def gather_kernel(table_hbm, idx_hbm, out_hbm, idx_v, rows_v, gsem, osem):
        wid = lax.axis_index("s") * info.num_cores + lax.axis_index("c")
        base = wid * b_per_w
        pltpu.sync_copy(idx_hbm.at[pl.ds(base, b_per_w)], idx_v)

        def _dst(c):
            jj = base + _CK * c
            f = jj // batch
            rem = jj % batch
            row0 = f * (batch // 4) + (rem // _BB) * (_BB // 4)
            kg = (rem % _BB) // _CK
            return out_hbm.at[pl.ds(row0, _CK), pl.ds(d * kg, d)]

        gathers = [None, None]
        outs = [None, None]
        gathers[0] = pltpu.async_copy(
            table_hbm.at[idx_v.at[pl.ds(0, _CK)]], rows_v.at[0], gsem
        )
        for c in range(n_chunks):
            cur = c % 2
            nxt = (c + 1) % 2
            if c + 1 < n_chunks:
                gathers[nxt] = pltpu.async_copy(
                    table_hbm.at[idx_v.at[pl.ds((c + 1) * _CK, _CK)]],
                    rows_v.at[nxt],
                    gsem,
                )
            gathers[cur].wait()
            if outs[cur] is not None:
                outs[cur].wait()
            outs[cur] = pltpu.async_copy(rows_v.at[cur], _dst(c), osem)
        for o in outs:
            if o is not None:
                o.wait()

    return gather_kernel


def kernel(inputs, indices):
    v, d = inputs.shape
    batch, n_fields = indices.shape
    table_packed = _tc_transpose_table(inputs.T)
    table_rm = table_packed.reshape(table_packed.size // d, d)
    idx = indices.T.reshape(-1)
    vv = jnp.bitwise_and(idx, _BW - 1)
    idxg = (idx - vv) + 4 * jnp.bitwise_and(vv, _BW // 4 - 1) + (
        vv >> (_BW // 4).bit_length() - 1
    )
    gathered = _make_sc_gather(idx.shape[0], batch, d)(table_rm, idxg)
    out_t = _tc_detranspose_out(gathered, n_fields, batch, d)
    return out_t.transpose(2, 0, 1)


# speedup vs baseline: 1.9679x; 1.0788x over previous
import functools

import jax
import jax.numpy as jnp
from jax import lax
from jax.experimental import pallas as pl
from jax.experimental.pallas import tpu as pltpu
from jax.experimental.pallas import tpu_sc as plsc

_BW = 8192
_BB = 2048
_CK = 512


def _pack4_transpose_kernel(x_ref, o_ref):
    q = x_ref.shape[1] // 2
    d = x_ref.shape[0]
    y = x_ref[...].T
    for k in range(2):
        o_ref[:, d * k : d * (k + 1)] = y[q * k : q * (k + 1), :]


def _tc_transpose_table(tt):
    d, v = tt.shape
    nb = pl.cdiv(v, _BW)
    return pl.pallas_call(
        _pack4_transpose_kernel,
        grid=(nb,),
        in_specs=[pl.BlockSpec((d, _BW), lambda i: (0, i))],
        out_specs=pl.BlockSpec((_BW // 2, 128), lambda i: (i, 0)),
        out_shape=jax.ShapeDtypeStruct((nb * _BW // 2, 128), jnp.float32),
    )(tt)


def _unpack4_transpose_kernel(x_ref, o_ref):
    d = o_ref.shape[1]
    q = x_ref.shape[0]
    eye = jnp.eye(d, dtype=x_ref.dtype)
    for k in range(4):
        o_ref[0, :, q * k : q * (k + 1)] = lax.dot_general(
            eye,
            x_ref[:, d * k : d * (k + 1)],
            (((1,), (1,)), ((), ())),
            preferred_element_type=jnp.float32,
        )


def _tc_detranspose_out(g2, f, b, d):
    nb = b // _BB
    return pl.pallas_call(
        _unpack4_transpose_kernel,
        grid=(f, nb),
        in_specs=[
            pl.BlockSpec((_BB // 4, 128), lambda i, j, nb=nb: (i * nb + j, 0))
        ],
        out_specs=pl.BlockSpec((1, d, _BB), lambda i, j: (i, 0, j)),
        out_shape=jax.ShapeDtypeStruct((f, d, b), jnp.float32),
    )(g2)


def _make_sc_gather(b_total: int, batch: int, d: int):
    info = plsc.get_sparse_core_info()
    nw = info.num_cores * info.num_subcores
    b_per_w = b_total // nw
    n_chunks = b_per_w // _CK

    mesh = plsc.VectorSubcoreMesh(core_axis_name="c", subcore_axis_name="s")

    @functools.partial(
        pl.kernel,
        mesh=mesh,
        out_type=jax.ShapeDtypeStruct((b_total // 4, 128), jnp.float32),
        scratch_types=[
            pltpu.VMEM((b_per_w,), jnp.int32),
            pltpu.VMEM((2, _CK, d), jnp.float32),
            pltpu.SemaphoreType.DMA,
            pltpu.SemaphoreType.DMA,
        ],
        compiler_params=pltpu.CompilerParams(use_tc_tiling_on_sc=False),
    )
    def gather_kernel(table_hbm, idx_hbm, out_hbm, idx_v, rows_v, gsem, osem):
        wid = lax.axis_index("s") * info.num_cores + lax.axis_index("c")
        base = wid * b_per_w
        pltpu.sync_copy(idx_hbm.at[pl.ds(base, b_per_w)], idx_v)

        def _dst(c):
            jj = base + _CK * c
            f = jj // batch
            rem = jj % batch
            row0 = f * (batch // 4) + (rem // _BB) * (_BB // 4)
            kg = (rem % _BB) // _CK
            return out_hbm.at[pl.ds(row0, _CK), pl.ds(d * kg, d)]

        gathers = [None, None]
        outs = [None, None]
        gathers[0] = pltpu.async_copy(
            table_hbm.at[idx_v.at[pl.ds(0, _CK)]], rows_v.at[0], gsem
        )
        for c in range(n_chunks):
            cur = c % 2
            nxt = (c + 1) % 2
            if c + 1 < n_chunks:
                gathers[nxt] = pltpu.async_copy(
                    table_hbm.at[idx_v.at[pl.ds((c + 1) * _CK, _CK)]],
                    rows_v.at[nxt],
                    gsem,
                )
            gathers[cur].wait()
            if outs[cur] is not None:
                outs[cur].wait()
            outs[cur] = pltpu.async_copy(rows_v.at[cur], _dst(c), osem)
        for o in outs:
            if o is not None:
                o.wait()

    return gather_kernel


def kernel(inputs, indices):
    v, d = inputs.shape
    batch, n_fields = indices.shape
    table_packed = _tc_transpose_table(inputs.T)
    table_rm = table_packed.reshape(table_packed.size // d, d)
    idx = indices.T.reshape(-1)
    vv = jnp.bitwise_and(idx, _BW - 1)
    idxg = 2 * (idx - vv) + 4 * jnp.bitwise_and(vv, _BW // 2 - 1) + (
        vv >> (_BW // 2).bit_length() - 1
    )
    gathered = _make_sc_gather(idx.shape[0], batch, d)(table_rm, idxg)
    out_t = _tc_detranspose_out(gathered, n_fields, batch, d)
    return out_t.transpose(2, 0, 1)


# speedup vs baseline: 2.2102x; 1.1231x over previous
import functools

import jax
import jax.numpy as jnp
from jax import lax
from jax.experimental import pallas as pl
from jax.experimental.pallas import tpu as pltpu
from jax.experimental.pallas import tpu_sc as plsc

_BW = 8192
_BB = 8192
_CK = 1024


def _pack4_transpose_kernel(x_ref, o_ref):
    q = x_ref.shape[1] // 2
    d = x_ref.shape[0]
    y = x_ref[...].T
    for k in range(2):
        o_ref[:, d * k : d * (k + 1)] = y[q * k : q * (k + 1), :]


def _tc_transpose_table(tt):
    d, v = tt.shape
    nb = pl.cdiv(v, _BW)
    return pl.pallas_call(
        _pack4_transpose_kernel,
        grid=(nb,),
        in_specs=[pl.BlockSpec((d, _BW), lambda i: (0, i))],
        out_specs=pl.BlockSpec((_BW // 2, 128), lambda i: (i, 0)),
        out_shape=jax.ShapeDtypeStruct((nb * _BW // 2, 128), jnp.float32),
    )(tt)


def _unpack4_transpose_kernel(x_ref, o_ref):
    d = o_ref.shape[1]
    q = x_ref.shape[0]
    for k in range(4):
        o_ref[0, :, q * k : q * (k + 1)] = x_ref[:, d * k : d * (k + 1)].T


def _tc_detranspose_out(g2, f, b, d):
    nb = b // _BB
    return pl.pallas_call(
        _unpack4_transpose_kernel,
        grid=(f, nb),
        in_specs=[
            pl.BlockSpec((_BB // 4, 128), lambda i, j, nb=nb: (i * nb + j, 0))
        ],
        out_specs=pl.BlockSpec((1, d, _BB), lambda i, j: (i, 0, j)),
        out_shape=jax.ShapeDtypeStruct((f, d, b), jnp.float32),
    )(g2)


def _make_sc_gather(b_total: int, batch: int, d: int):
    info = plsc.get_sparse_core_info()
    nw = info.num_cores * info.num_subcores
    b_per_w = b_total // nw
    n_chunks = b_per_w // _CK

    mesh = plsc.VectorSubcoreMesh(core_axis_name="c", subcore_axis_name="s")

    @functools.partial(
        pl.kernel,
        mesh=mesh,
        out_type=jax.ShapeDtypeStruct((b_total // 4, 128), jnp.float32),
        scratch_types=[
            pltpu.VMEM((b_per_w,), jnp.int32),
            pltpu.VMEM((2, _CK, d), jnp.float32),
            pltpu.SemaphoreType.DMA,
            pltpu.SemaphoreType.DMA,
        ],
        compiler_params=pltpu.CompilerParams(use_tc_tiling_on_sc=False),
    )
    def gather_kernel(table_hbm, idx_hbm, out_hbm, idx_v, rows_v, gsem, osem):
        wid = lax.axis_index("s") * info.num_cores + lax.axis_index("c")
        base = wid * b_per_w
        pltpu.sync_copy(idx_hbm.at[pl.ds(base, b_per_w)], idx_v)

        def _dst(c):
            jj = base + _CK * c
            f = jj // batch
            rem = jj % batch
            q = _BB // 4
            kg = (rem % _BB) // q
            row0 = f * (batch // 4) + (rem // _BB) * q + rem % q
            return out_hbm.at[pl.ds(row0, _CK), pl.ds(d * kg, d)]

        gathers = [None, None]
        outs = [None, None]
        gathers[0] = pltpu.async_copy(
            table_hbm.at[idx_v.at[pl.ds(0, _CK)]], rows_v.at[0], gsem
        )
        for c in range(n_chunks):
            cur = c % 2
            nxt = (c + 1) % 2
            if c + 1 < n_chunks:
                gathers[nxt] = pltpu.async_copy(
                    table_hbm.at[idx_v.at[pl.ds((c + 1) * _CK, _CK)]],
                    rows_v.at[nxt],
                    gsem,
                )
            gathers[cur].wait()
            if outs[cur] is not None:
                outs[cur].wait()
            outs[cur] = pltpu.async_copy(rows_v.at[cur], _dst(c), osem)
        for o in outs:
            if o is not None:
                o.wait()

    return gather_kernel


def kernel(inputs, indices):
    v, d = inputs.shape
    batch, n_fields = indices.shape
    table_packed = _tc_transpose_table(inputs.T)
    table_rm = table_packed.reshape(table_packed.size // d, d)
    idx = indices.T.reshape(-1)
    vv = jnp.bitwise_and(idx, _BW - 1)
    idxg = 2 * (idx - vv) + 4 * jnp.bitwise_and(vv, _BW // 2 - 1) + (
        vv >> (_BW // 2).bit_length() - 1
    )
    gathered = _make_sc_gather(idx.shape[0], batch, d)(table_rm, idxg)
    out_t = _tc_detranspose_out(gathered, n_fields, batch, d)
    return out_t.transpose(2, 0, 1)


# speedup vs baseline: 2.3487x; 1.0627x over previous
import functools

import jax
import jax.numpy as jnp
from jax import lax
from jax.experimental import pallas as pl
from jax.experimental.pallas import tpu as pltpu
from jax.experimental.pallas import tpu_sc as plsc

_BW = 16384
_BB = 16384
_CK = 1024


def _pack4_transpose_kernel(x_ref, o_ref):
    q = x_ref.shape[1] // 2
    d = x_ref.shape[0]
    y = x_ref[...].T
    for k in range(2):
        o_ref[:, d * k : d * (k + 1)] = y[q * k : q * (k + 1), :]


def _tc_transpose_table(tt):
    d, v = tt.shape
    nb = pl.cdiv(v, _BW)
    return pl.pallas_call(
        _pack4_transpose_kernel,
        grid=(nb,),
        in_specs=[pl.BlockSpec((d, _BW), lambda i: (0, i))],
        out_specs=pl.BlockSpec((_BW // 2, 128), lambda i: (i, 0)),
        out_shape=jax.ShapeDtypeStruct((nb * _BW // 2, 128), jnp.float32),
    )(tt)


def _unpack4_transpose_kernel(x_ref, o_ref):
    d = o_ref.shape[1]
    q = x_ref.shape[0]
    for k in range(4):
        o_ref[0, :, q * k : q * (k + 1)] = x_ref[:, d * k : d * (k + 1)].T


def _tc_detranspose_out(g2, f, b, d):
    nb = b // _BB
    return pl.pallas_call(
        _unpack4_transpose_kernel,
        grid=(f, nb),
        in_specs=[
            pl.BlockSpec((_BB // 4, 128), lambda i, j, nb=nb: (i * nb + j, 0))
        ],
        out_specs=pl.BlockSpec((1, d, _BB), lambda i, j: (i, 0, j)),
        out_shape=jax.ShapeDtypeStruct((f, d, b), jnp.float32),
    )(g2)


def _make_sc_gather(b_total: int, batch: int, d: int):
    info = plsc.get_sparse_core_info()
    nw = info.num_cores * info.num_subcores
    b_per_w = b_total // nw
    n_chunks = b_per_w // _CK

    mesh = plsc.VectorSubcoreMesh(core_axis_name="c", subcore_axis_name="s")

    @functools.partial(
        pl.kernel,
        mesh=mesh,
        out_type=jax.ShapeDtypeStruct((b_total // 4, 128), jnp.float32),
        scratch_types=[
            pltpu.VMEM((b_per_w,), jnp.int32),
            pltpu.VMEM((2, _CK, d), jnp.float32),
            pltpu.SemaphoreType.DMA,
            pltpu.SemaphoreType.DMA,
        ],
        compiler_params=pltpu.CompilerParams(use_tc_tiling_on_sc=False),
    )
    def gather_kernel(table_hbm, idx_hbm, out_hbm, idx_v, rows_v, gsem, osem):
        wid = lax.axis_index("s") * info.num_cores + lax.axis_index("c")
        base = wid * b_per_w
        pltpu.sync_copy(idx_hbm.at[pl.ds(base, b_per_w)], idx_v)

        def _dst(c):
            jj = base + _CK * c
            f = jj // batch
            rem = jj % batch
            q = _BB // 4
            kg = (rem % _BB) // q
            row0 = f * (batch // 4) + (rem // _BB) * q + rem % q
            return out_hbm.at[pl.ds(row0, _CK), pl.ds(d * kg, d)]

        gathers = [None, None]
        outs = [None, None]
        gathers[0] = pltpu.async_copy(
            table_hbm.at[idx_v.at[pl.ds(0, _CK)]], rows_v.at[0], gsem
        )
        for c in range(n_chunks):
            cur = c % 2
            nxt = (c + 1) % 2
            if c + 1 < n_chunks:
                gathers[nxt] = pltpu.async_copy(
                    table_hbm.at[idx_v.at[pl.ds((c + 1) * _CK, _CK)]],
                    rows_v.at[nxt],
                    gsem,
                )
            gathers[cur].wait()
            if outs[cur] is not None:
                outs[cur].wait()
            outs[cur] = pltpu.async_copy(rows_v.at[cur], _dst(c), osem)
        for o in outs:
            if o is not None:
                o.wait()

    return gather_kernel


def kernel(inputs, indices):
    v, d = inputs.shape
    batch, n_fields = indices.shape
    table_packed = _tc_transpose_table(inputs.T)
    table_rm = table_packed.reshape(table_packed.size // d, d)
    idx = indices.T.reshape(-1)
    vv = jnp.bitwise_and(idx, _BW - 1)
    idxg = 2 * (idx - vv) + 4 * jnp.bitwise_and(vv, _BW // 2 - 1) + (
        vv >> (_BW // 2).bit_length() - 1
    )
    gathered = _make_sc_gather(idx.shape[0], batch, d)(table_rm, idxg)
    out_t = _tc_detranspose_out(gathered, n_fields, batch, d)
    return out_t.transpose(2, 0, 1)


# speedup vs baseline: 2.3585x; 1.0042x over previous
import functools

import jax
import jax.numpy as jnp
from jax import lax
from jax.experimental import pallas as pl
from jax.experimental.pallas import tpu as pltpu
from jax.experimental.pallas import tpu_sc as plsc

_BW = 32768
_BB = 16384
_CK = 1024


def _pack2_transpose_kernel(x_ref, o_ref):
    q = x_ref.shape[1] // 2
    d = x_ref.shape[0]
    y = x_ref[...].T
    for k in range(2):
        o_ref[:, d * k : d * (k + 1)] = y[q * k : q * (k + 1), :]


def _tc_transpose_table(tt):
    d, v = tt.shape
    nb = pl.cdiv(v, _BW)
    return pl.pallas_call(
        _pack2_transpose_kernel,
        grid=(nb,),
        in_specs=[pl.BlockSpec((d, _BW), lambda i: (0, i))],
        out_specs=pl.BlockSpec((_BW // 2, 128), lambda i: (i, 0)),
        out_shape=jax.ShapeDtypeStruct((nb * _BW // 2, 128), jnp.float32),
    )(tt)


def _unpack4_transpose_kernel(x_ref, o_ref):
    d = o_ref.shape[1]
    q = x_ref.shape[0]
    for k in range(4):
        o_ref[0, :, q * k : q * (k + 1)] = x_ref[:, d * k : d * (k + 1)].T


def _tc_detranspose_out(g2, f, b, d):
    nb = b // _BB
    return pl.pallas_call(
        _unpack4_transpose_kernel,
        grid=(f, nb),
        in_specs=[
            pl.BlockSpec((_BB // 4, 128), lambda i, j, nb=nb: (i * nb + j, 0))
        ],
        out_specs=pl.BlockSpec((1, d, _BB), lambda i, j: (i, 0, j)),
        out_shape=jax.ShapeDtypeStruct((f, d, b), jnp.float32),
    )(g2)


def _make_sc_gather(b_total: int, batch: int, d: int):
    info = plsc.get_sparse_core_info()
    nw = info.num_cores * info.num_subcores
    b_per_w = b_total // nw
    n_chunks = b_per_w // _CK

    mesh = plsc.VectorSubcoreMesh(core_axis_name="c", subcore_axis_name="s")

    @functools.partial(
        pl.kernel,
        mesh=mesh,
        out_type=jax.ShapeDtypeStruct((b_total // 4, 128), jnp.float32),
        scratch_types=[
            pltpu.VMEM((b_per_w,), jnp.int32),
            pltpu.VMEM((2, _CK, d), jnp.float32),
            pltpu.SemaphoreType.DMA,
            pltpu.SemaphoreType.DMA,
        ],
        compiler_params=pltpu.CompilerParams(use_tc_tiling_on_sc=False),
    )
    def gather_kernel(table_hbm, idx_hbm, out_hbm, idx_v, rows_v, gsem, osem):
        wid = lax.axis_index("s") * info.num_cores + lax.axis_index("c")
        base = wid * b_per_w
        pltpu.sync_copy(idx_hbm.at[pl.ds(base, b_per_w)], idx_v)

        def _dst(c):
            jj = base + _CK * c
            f = jj // batch
            rem = jj % batch
            q = _BB // 4
            kg = (rem % _BB) // q
            row0 = f * (batch // 4) + (rem // _BB) * q + rem % q
            return out_hbm.at[pl.ds(row0, _CK), pl.ds(d * kg, d)]

        gathers = [None, None]
        outs = [None, None]
        gathers[0] = pltpu.async_copy(
            table_hbm.at[idx_v.at[pl.ds(0, _CK)]], rows_v.at[0], gsem
        )
        for c in range(n_chunks):
            cur = c % 2
            nxt = (c + 1) % 2
            if c + 1 < n_chunks:
                gathers[nxt] = pltpu.async_copy(
                    table_hbm.at[idx_v.at[pl.ds((c + 1) * _CK, _CK)]],
                    rows_v.at[nxt],
                    gsem,
                )
            gathers[cur].wait()
            if outs[cur] is not None:
                outs[cur].wait()
            outs[cur] = pltpu.async_copy(rows_v.at[cur], _dst(c), osem)
        for o in outs:
            if o is not None:
                o.wait()

    return gather_kernel


def kernel(inputs, indices):
    v, d = inputs.shape
    batch, n_fields = indices.shape
    table_packed = _tc_transpose_table(inputs.T)
    table_rm = table_packed.reshape(table_packed.size // d, d)
    idx = indices.T.reshape(-1)
    vv = jnp.bitwise_and(idx, _BW - 1)
    idxg = 2 * (idx - vv) + 4 * jnp.bitwise_and(vv, _BW // 2 - 1) + (
        vv >> (_BW // 2).bit_length() - 1
    )
    gathered = _make_sc_gather(idx.shape[0], batch, d)(table_rm, idxg)
    out_t = _tc_detranspose_out(gathered, n_fields, batch, d)
    return out_t.transpose(2, 0, 1)
